# baseline (speedup 1.0000x reference)
import functools

import numpy as np

import jax
import jax.numpy as jnp
from jax import lax
from jax.experimental import pallas as pl
from jax.experimental.pallas import tpu as pltpu
from jax.experimental.pallas import tpu_sc as plsc

BATCH = 16384
EMB_K = 16

_NC, _NS = 2, 16
_NW = _NC * _NS
_BPW = BATCH // _NW
_CHB = 64
_NCH = _BPW // _CHB
_L = 16
_NBIAS_PAD = 800 * 128

_LANE = np.arange(_L, dtype=np.int32)


def _c(arr):
    return jnp.asarray(arr, dtype=jnp.int32)


@functools.cache
def _make_sc_gather():
    mesh = plsc.VectorSubcoreMesh(core_axis_name="c", subcore_axis_name="s")

    @functools.partial(
        pl.kernel,
        mesh=mesh,
        compiler_params=pltpu.CompilerParams(use_tc_tiling_on_sc=True,
                                             needs_layout_passes=False),
        out_type=[
            jax.ShapeDtypeStruct((BATCH // 8, 128), jnp.float32),
            jax.ShapeDtypeStruct((BATCH // 8, 128), jnp.float32),
            jax.ShapeDtypeStruct((BATCH,), jnp.float32),
        ],
        scratch_types=[
            pltpu.VMEM((_NCH, _CHB), jnp.int32),
            pltpu.VMEM((_NCH, _CHB), jnp.int32),
            pltpu.VMEM((_NCH, _CHB), jnp.int32),
            pltpu.VMEM((_NCH, _CHB), jnp.int32),
            pltpu.VMEM((_NCH, _CHB), jnp.int32),
            pltpu.VMEM((_NCH, _CHB), jnp.int32),
            pltpu.VMEM((_CHB, 128), jnp.float32),
            pltpu.VMEM((_CHB, 128), jnp.float32),
            pltpu.VMEM((_CHB, 128), jnp.float32),
            pltpu.VMEM((_CHB, 128), jnp.float32),
            pltpu.VMEM((_BPW // 8, 128), jnp.float32),
            pltpu.VMEM((_BPW // 8, 128), jnp.float32),
            pltpu.VMEM((_BPW,), jnp.float32),
            pltpu.SemaphoreType.DMA,
        ],
    )
    def gather_kernel(uidx_hbm, iidx_hbm, w_hbm, h_hbm, ub_hbm, ib_hbm,
                      uz_out, vz_out, bsum_out,
                      uidx_v, iidx_v, uer_v, ier_v, ubr_v, ibr_v,
                      wbuf, hbuf, ubbuf, ibbuf,
                      uz_v, vz_v, bsum_v, sem):
        wid = lax.axis_index("s") * _NC + lax.axis_index("c")
        base = wid * _BPW

        idx_cps = []
        for j in range(_NCH):
            idx_cps.append(pltpu.async_copy(
                uidx_hbm.at[pl.ds(base + j * _CHB, _CHB)], uidx_v.at[j], sem))
            idx_cps.append(pltpu.async_copy(
                iidx_hbm.at[pl.ds(base + j * _CHB, _CHB)], iidx_v.at[j], sem))
        for cp in idx_cps:
            cp.wait()

        for j in range(_NCH):
            for g in range(_CHB // _L):
                s = pl.ds(g * _L, _L)
                u = uidx_v[j, s]
                i = iidx_v[j, s]
                uer_v[j, s] = lax.shift_right_logical(u, 3)
                ier_v[j, s] = lax.shift_right_logical(i, 3)
                ubr_v[j, s] = lax.shift_right_logical(u, 7)
                ibr_v[j, s] = lax.shift_right_logical(i, 7)

        lane = lax.iota(jnp.int32, _L)
        prow_off = (lane >= 8).astype(jnp.int32)
        pcol0 = (lane & 7) * EMB_K
        for j in range(_NCH):
            cps = [
                pltpu.async_copy(w_hbm.at[uer_v.at[j]], wbuf, sem),
                pltpu.async_copy(h_hbm.at[ier_v.at[j]], hbuf, sem),
                pltpu.async_copy(ub_hbm.at[ubr_v.at[j]], ubbuf, sem),
                pltpu.async_copy(ib_hbm.at[ibr_v.at[j]], ibbuf, sem),
            ]
            for cp in cps:
                cp.wait()
            for g in range(_CHB // _L):
                s = pl.ds(g * _L, _L)
                rows = lane + (g * _L)
                u = uidx_v[j, s]
                i = iidx_v[j, s]
                ucol0 = (u & 7) * EMB_K
                icol0 = (i & 7) * EMB_K
                b16 = (j * _CHB + g * _L) * EMB_K
                prow = prow_off + (b16 // 128)
                for k in range(EMB_K):
                    pcol = pcol0 + k
                    ue = plsc.load_gather(wbuf, [rows, ucol0 + k])
                    ie = plsc.load_gather(hbuf, [rows, icol0 + k])
                    plsc.store_scatter(uz_v, [prow, pcol], ue)
                    plsc.store_scatter(vz_v, [prow, pcol], ie)
                ubias = plsc.load_gather(ubbuf, [rows, u & 127])
                ibias = plsc.load_gather(ibbuf, [rows, i & 127])
                bsum_v[pl.ds(j * _CHB + g * _L, _L)] = ubias + ibias

        obase = wid * (_BPW // 8)
        out_cps = [
            pltpu.async_copy(uz_v, uz_out.at[pl.ds(obase, _BPW // 8)], sem),
            pltpu.async_copy(vz_v, vz_out.at[pl.ds(obase, _BPW // 8)], sem),
            pltpu.async_copy(bsum_v, bsum_out.at[pl.ds(base, _BPW)], sem),
        ]
        for cp in out_cps:
            cp.wait()

    return gather_kernel


_BLK = 2048


def _mlp_body(uz_ref, vz_ref, bsum_ref, w1_ref, b1_ref, w2_ref, out_ref):
    uz = uz_ref[...]
    vz = vz_ref[...]
    w1 = w1_ref[...]
    h = lax.dot_general(uz, w1[:, :EMB_K], (((1,), (1,)), ((), ())),
                        preferred_element_type=jnp.float32)
    h = h + lax.dot_general(vz, w1[:, EMB_K:], (((1,), (1,)), ((), ())),
                            preferred_element_type=jnp.float32)
    h = jnp.maximum(h + b1_ref[...], 0.0)
    out = jnp.sum(h * w2_ref[...], axis=1, keepdims=True)
    out_ref[...] = out + bsum_ref[...]


def _mlp(uz, vz, bsum, w1, b1, w2):
    grid = (BATCH // _BLK,)
    row_blk = lambda i: (i, 0)
    w_blk = lambda i: (0, 0)
    return pl.pallas_call(
        _mlp_body,
        grid=grid,
        in_specs=[
            pl.BlockSpec((_BLK, EMB_K), row_blk),
            pl.BlockSpec((_BLK, EMB_K), row_blk),
            pl.BlockSpec((_BLK, 1), row_blk),
            pl.BlockSpec((EMB_K, 2 * EMB_K), w_blk),
            pl.BlockSpec((1, EMB_K), w_blk),
            pl.BlockSpec((1, EMB_K), w_blk),
        ],
        out_specs=pl.BlockSpec((_BLK, 1), row_blk),
        out_shape=jax.ShapeDtypeStruct((BATCH, 1), jnp.float32),
    )(uz, vz, bsum, w1, b1, w2)


def kernel(x, W, H, lin1_w, lin1_b, lin2_w, user_bias, item_bias):
    uidx = x[:, 0]
    iidx = x[:, 1]
    w128 = W.reshape(-1, 128)
    h128 = H.reshape(-1, 128)
    ubp = jnp.pad(user_bias.reshape(-1), (0, _NBIAS_PAD - user_bias.shape[0]))
    ibp = jnp.pad(item_bias.reshape(-1), (0, _NBIAS_PAD - item_bias.shape[0]))
    uzp, vzp, bsum = _make_sc_gather()(
        uidx, iidx, w128, h128, ubp.reshape(-1, 128), ibp.reshape(-1, 128))
    uz = uzp.reshape(BATCH, EMB_K)
    vz = vzp.reshape(BATCH, EMB_K)
    return _mlp(uz, vz, bsum.reshape(BATCH, 1), lin1_w,
                lin1_b.reshape(1, EMB_K), lin2_w)


# speedup vs baseline: 1.1501x; 1.1501x over previous
import functools

import jax
import jax.numpy as jnp
from jax import lax
from jax.experimental import pallas as pl
from jax.experimental.pallas import tpu as pltpu
from jax.experimental.pallas import tpu_sc as plsc

BATCH = 16384
EMB_K = 16

_NC, _NS = 2, 16
_NW = _NC * _NS
_BPW = BATCH // _NW
_CHB = 128
_NCH = _BPW // _CHB
_L = 16


@functools.cache
def _make_sc_gather():
    mesh = plsc.VectorSubcoreMesh(core_axis_name="c", subcore_axis_name="s")

    @functools.partial(
        pl.kernel,
        mesh=mesh,
        compiler_params=pltpu.CompilerParams(use_tc_tiling_on_sc=False,
                                             needs_layout_passes=False),
        out_type=[
            jax.ShapeDtypeStruct((BATCH, EMB_K), jnp.float32),
            jax.ShapeDtypeStruct((BATCH, EMB_K), jnp.float32),
            jax.ShapeDtypeStruct((BATCH,), jnp.float32),
        ],
        scratch_types=[
            pltpu.VMEM((2 * _BPW,), jnp.int32),
            pltpu.VMEM((_NCH, _CHB), jnp.int32),
            pltpu.VMEM((_NCH, _CHB), jnp.int32),
            pltpu.VMEM((_NCH, _CHB), jnp.int32),
            pltpu.VMEM((_NCH, _CHB), jnp.int32),
            pltpu.VMEM((_BPW, EMB_K), jnp.float32),
            pltpu.VMEM((_BPW, EMB_K), jnp.float32),
            pltpu.VMEM((_BPW, _L), jnp.float32),
            pltpu.VMEM((_BPW, _L), jnp.float32),
            pltpu.VMEM((_BPW,), jnp.float32),
            pltpu.SemaphoreType.DMA,
        ],
    )
    def gather_kernel(xflat_hbm, w_hbm, h_hbm, ub_hbm, ib_hbm,
                      uz_out, vz_out, bsum_out,
                      xv, uidx_v, iidx_v, uhi_v, ihi_v,
                      urows_v, vrows_v, ubr_v, ibr_v, bsum_v, sem):
        wid = lax.axis_index("s") * _NC + lax.axis_index("c")
        base = wid * _BPW

        pltpu.async_copy(
            xflat_hbm.at[pl.ds(2 * base, 2 * _BPW)], xv, sem).wait()

        lane = lax.iota(jnp.int32, _L)
        lane2 = lane * 2
        for j in range(_NCH):
            for g in range(_CHB // _L):
                s = pl.ds(g * _L, _L)
                off = (j * _CHB + g * _L) * 2
                u = plsc.load_gather(xv, [lane2 + off])
                i = plsc.load_gather(xv, [lane2 + (off + 1)])
                uidx_v[j, s] = u
                iidx_v[j, s] = i
                uhi_v[j, s] = lax.shift_right_logical(u, 4)
                ihi_v[j, s] = lax.shift_right_logical(i, 4)

        cps = []
        for j in range(_NCH):
            r = pl.ds(j * _CHB, _CHB)
            cps.append(pltpu.async_copy(w_hbm.at[uidx_v.at[j]], urows_v.at[r], sem))
            cps.append(pltpu.async_copy(h_hbm.at[iidx_v.at[j]], vrows_v.at[r], sem))
            cps.append(pltpu.async_copy(ub_hbm.at[uhi_v.at[j]], ubr_v.at[r], sem))
            cps.append(pltpu.async_copy(ib_hbm.at[ihi_v.at[j]], ibr_v.at[r], sem))
        for cp in cps:
            cp.wait()

        for j in range(_NCH):
            for g in range(_CHB // _L):
                s = pl.ds(g * _L, _L)
                rows = lane + (j * _CHB + g * _L)
                ub_e = plsc.load_gather(ubr_v, [rows, uidx_v[j, s] & 15])
                ib_e = plsc.load_gather(ibr_v, [rows, iidx_v[j, s] & 15])
                bsum_v[pl.ds(j * _CHB + g * _L, _L)] = ub_e + ib_e

        out_cps = [
            pltpu.async_copy(urows_v, uz_out.at[pl.ds(base, _BPW)], sem),
            pltpu.async_copy(vrows_v, vz_out.at[pl.ds(base, _BPW)], sem),
            pltpu.async_copy(bsum_v, bsum_out.at[pl.ds(base, _BPW)], sem),
        ]
        for cp in out_cps:
            cp.wait()

    return gather_kernel


_BLK = 2048


def _mlp_body(uz_ref, vz_ref, bsum_ref, w1_ref, b1_ref, w2_ref, out_ref):
    uz = uz_ref[...]
    vz = vz_ref[...]
    w1 = w1_ref[...]
    h = lax.dot_general(uz, w1[:, :EMB_K], (((1,), (1,)), ((), ())),
                        preferred_element_type=jnp.float32)
    h = h + lax.dot_general(vz, w1[:, EMB_K:], (((1,), (1,)), ((), ())),
                            preferred_element_type=jnp.float32)
    h = jnp.maximum(h + b1_ref[...], 0.0)
    out = jnp.sum(h * w2_ref[...], axis=1, keepdims=True)
    out_ref[...] = out + bsum_ref[...]


def _mlp(uz, vz, bsum, w1, b1, w2):
    grid = (BATCH // _BLK,)
    row_blk = lambda i: (i, 0)
    w_blk = lambda i: (0, 0)
    return pl.pallas_call(
        _mlp_body,
        grid=grid,
        in_specs=[
            pl.BlockSpec((_BLK, EMB_K), row_blk),
            pl.BlockSpec((_BLK, EMB_K), row_blk),
            pl.BlockSpec((_BLK, 1), row_blk),
            pl.BlockSpec((EMB_K, 2 * EMB_K), w_blk),
            pl.BlockSpec((1, EMB_K), w_blk),
            pl.BlockSpec((1, EMB_K), w_blk),
        ],
        out_specs=pl.BlockSpec((_BLK, 1), row_blk),
        out_shape=jax.ShapeDtypeStruct((BATCH, 1), jnp.float32),
    )(uz, vz, bsum, w1, b1, w2)


def kernel(x, W, H, lin1_w, lin1_b, lin2_w, user_bias, item_bias):
    xflat = x.reshape(-1)
    ub16 = user_bias.reshape(-1, _L)
    ib16 = item_bias.reshape(-1, _L)
    uz, vz, bsum = _make_sc_gather()(xflat, W, H, ub16, ib16)
    return _mlp(uz, vz, bsum.reshape(BATCH, 1), lin1_w,
                lin1_b.reshape(1, EMB_K), lin2_w)


# speedup vs baseline: 1.2197x; 1.0605x over previous
import functools

import jax
import jax.numpy as jnp
from jax import lax
from jax.experimental import pallas as pl
from jax.experimental.pallas import tpu as pltpu
from jax.experimental.pallas import tpu_sc as plsc

BATCH = 16384
EMB_K = 16

_NC, _NS = 2, 16
_NW = _NC * _NS
_BPW = BATCH // _NW
_CHB = 128
_NCH = _BPW // _CHB
_L = 16


@functools.cache
def _make_sc_gather():
    mesh = plsc.VectorSubcoreMesh(core_axis_name="c", subcore_axis_name="s")

    @functools.partial(
        pl.kernel,
        mesh=mesh,
        compiler_params=pltpu.CompilerParams(use_tc_tiling_on_sc=False,
                                             needs_layout_passes=False),
        out_type=[
            jax.ShapeDtypeStruct((BATCH, EMB_K), jnp.float32),
            jax.ShapeDtypeStruct((BATCH, EMB_K), jnp.float32),
            jax.ShapeDtypeStruct((BATCH,), jnp.float32),
        ],
        scratch_types=[
            pltpu.VMEM((_NCH, _CHB), jnp.int32),
            pltpu.VMEM((_NCH, _CHB), jnp.int32),
            pltpu.VMEM((_BPW, EMB_K), jnp.float32),
            pltpu.VMEM((_BPW, EMB_K), jnp.float32),
            pltpu.VMEM((_BPW,), jnp.float32),
            pltpu.VMEM((_BPW,), jnp.float32),
            pltpu.VMEM((_BPW,), jnp.float32),
            pltpu.SemaphoreType.DMA,
        ],
    )
    def gather_kernel(xt_hbm, w_hbm, h_hbm, ub_hbm, ib_hbm,
                      uz_out, vz_out, bsum_out,
                      uidx_v, iidx_v, urows_v, vrows_v, ubr_v, ibr_v,
                      bsum_v, sem):
        wid = lax.axis_index("s") * _NC + lax.axis_index("c")
        base = wid * _BPW

        idx_cps = []
        for j in range(_NCH):
            idx_cps.append(pltpu.async_copy(
                xt_hbm.at[pl.ds(base + j * _CHB, _CHB)], uidx_v.at[j], sem))
            idx_cps.append(pltpu.async_copy(
                xt_hbm.at[pl.ds(BATCH + base + j * _CHB, _CHB)],
                iidx_v.at[j], sem))
        for cp in idx_cps:
            cp.wait()

        cps = []
        for j in range(_NCH):
            r = pl.ds(j * _CHB, _CHB)
            cps.append(pltpu.async_copy(w_hbm.at[uidx_v.at[j]], urows_v.at[r], sem))
            cps.append(pltpu.async_copy(h_hbm.at[iidx_v.at[j]], vrows_v.at[r], sem))
            cps.append(pltpu.async_copy(ub_hbm.at[uidx_v.at[j]], ubr_v.at[r], sem))
            cps.append(pltpu.async_copy(ib_hbm.at[iidx_v.at[j]], ibr_v.at[r], sem))
        for cp in cps:
            cp.wait()

        for g in range(_BPW // _L):
            s = pl.ds(g * _L, _L)
            bsum_v[s] = ubr_v[s] + ibr_v[s]

        out_cps = [
            pltpu.async_copy(urows_v, uz_out.at[pl.ds(base, _BPW)], sem),
            pltpu.async_copy(vrows_v, vz_out.at[pl.ds(base, _BPW)], sem),
            pltpu.async_copy(bsum_v, bsum_out.at[pl.ds(base, _BPW)], sem),
        ]
        for cp in out_cps:
            cp.wait()

    return gather_kernel


_BLK = 2048


def _mlp_body(uz_ref, vz_ref, bsum_ref, w1_ref, b1_ref, w2_ref, out_ref):
    uz = uz_ref[...]
    vz = vz_ref[...]
    w1 = w1_ref[...]
    h = lax.dot_general(uz, w1[:, :EMB_K], (((1,), (1,)), ((), ())),
                        preferred_element_type=jnp.float32)
    h = h + lax.dot_general(vz, w1[:, EMB_K:], (((1,), (1,)), ((), ())),
                            preferred_element_type=jnp.float32)
    h = jnp.maximum(h + b1_ref[...], 0.0)
    out = jnp.sum(h * w2_ref[...], axis=1, keepdims=True)
    out_ref[...] = out + bsum_ref[...]


def _mlp(uz, vz, bsum, w1, b1, w2):
    grid = (BATCH // _BLK,)
    row_blk = lambda i: (i, 0)
    w_blk = lambda i: (0, 0)
    return pl.pallas_call(
        _mlp_body,
        grid=grid,
        in_specs=[
            pl.BlockSpec((_BLK, EMB_K), row_blk),
            pl.BlockSpec((_BLK, EMB_K), row_blk),
            pl.BlockSpec((_BLK, 1), row_blk),
            pl.BlockSpec((EMB_K, 2 * EMB_K), w_blk),
            pl.BlockSpec((1, EMB_K), w_blk),
            pl.BlockSpec((1, EMB_K), w_blk),
        ],
        out_specs=pl.BlockSpec((_BLK, 1), row_blk),
        out_shape=jax.ShapeDtypeStruct((BATCH, 1), jnp.float32),
    )(uz, vz, bsum, w1, b1, w2)


def kernel(x, W, H, lin1_w, lin1_b, lin2_w, user_bias, item_bias):
    xt = x.T.reshape(-1)
    ubf = user_bias.reshape(-1)
    ibf = item_bias.reshape(-1)
    uz, vz, bsum = _make_sc_gather()(xt, W, H, ubf, ibf)
    return _mlp(uz, vz, bsum.reshape(BATCH, 1), lin1_w,
                lin1_b.reshape(1, EMB_K), lin2_w)


# speedup vs baseline: 2.3271x; 1.9080x over previous
import functools

import jax
import jax.numpy as jnp
from jax import lax
from jax.experimental import pallas as pl
from jax.experimental.pallas import tpu as pltpu
from jax.experimental.pallas import tpu_sc as plsc

BATCH = 16384
EMB_K = 16

_NC, _NS = 2, 16
_BPT = BATCH // _NS
_CHB = 128
_NCH = _BPT // _CHB
_L = 16
_TW = 100096


@functools.cache
def _make_sc_gather():
    mesh = plsc.VectorSubcoreMesh(core_axis_name="c", subcore_axis_name="s")

    @functools.partial(
        pl.kernel,
        mesh=mesh,
        compiler_params=pltpu.CompilerParams(use_tc_tiling_on_sc=False,
                                             needs_layout_passes=False),
        out_type=[
            jax.ShapeDtypeStruct((EMB_K, BATCH), jnp.float32),
            jax.ShapeDtypeStruct((EMB_K, BATCH), jnp.float32),
            jax.ShapeDtypeStruct((BATCH,), jnp.float32),
            jax.ShapeDtypeStruct((BATCH,), jnp.float32),
        ],
        scratch_types=[
            pltpu.VMEM_SHARED((EMB_K * _TW,), jnp.float32),
            pltpu.VMEM((_BPT,), jnp.int32),
            pltpu.VMEM((EMB_K, _CHB), jnp.int32),
            pltpu.VMEM((EMB_K, _BPT), jnp.float32),
            pltpu.VMEM((_BPT,), jnp.float32),
            pltpu.SemaphoreType.DMA,
        ],
    )
    def gather_kernel(xt_hbm, wt_hbm, ht_hbm, ub_hbm, ib_hbm,
                      uzt_out, vzt_out, ubg_out, ibg_out,
                      spm, idx_v, idxk_v, zt_v, br_v, sem):
        cid = lax.axis_index("c")
        sid = lax.axis_index("s")

        def side(tab_hbm, bias_hbm, xoff, zt_out, bg_out):
            pltpu.sync_copy(tab_hbm.at[sid],
                            spm.at[pl.ds(sid * _TW, _TW)])
            plsc.subcore_barrier()

            gbase = sid * _BPT
            pltpu.async_copy(
                xt_hbm.at[pl.ds(xoff + gbase, _BPT)], idx_v, sem).wait()

            def chunk(j, carry):
                r = pl.ds(j * _CHB, _CHB)
                bias_cp = pltpu.async_copy(
                    bias_hbm.at[idx_v.at[r]], br_v.at[r], sem)
                for k in range(EMB_K):
                    for g in range(_CHB // _L):
                        s = pl.ds(g * _L, _L)
                        idxk_v[k, s] = idx_v[pl.ds(j * _CHB + g * _L, _L)] \
                            + (k * _TW)
                cps = [pltpu.async_copy(
                    spm.at[idxk_v.at[k]],
                    zt_v.at[k, pl.ds(j * _CHB, _CHB)], sem)
                    for k in range(EMB_K)]
                for cp in cps:
                    cp.wait()
                bias_cp.wait()
                return carry

            lax.fori_loop(0, _NCH, chunk, 0)

            out_cps = [
                pltpu.async_copy(zt_v, zt_out.at[:, pl.ds(gbase, _BPT)], sem),
                pltpu.async_copy(br_v, bg_out.at[pl.ds(gbase, _BPT)], sem),
            ]
            for cp in out_cps:
                cp.wait()

        @pl.when(cid == 0)
        def _():
            side(wt_hbm, ub_hbm, 0, uzt_out, ubg_out)

        @pl.when(cid == 1)
        def _():
            side(ht_hbm, ib_hbm, BATCH, vzt_out, ibg_out)

    return gather_kernel


_BLK = 4096


def _mlp_body(uzt_ref, vzt_ref, ub_ref, ib_ref, w1_ref, b1_ref, w2_ref,
              out_ref):
    uzt = uzt_ref[...]
    vzt = vzt_ref[...]
    w1 = w1_ref[...]
    h = lax.dot_general(w1[:, :EMB_K], uzt, (((1,), (0,)), ((), ())),
                        preferred_element_type=jnp.float32)
    h = h + lax.dot_general(w1[:, EMB_K:], vzt, (((1,), (0,)), ((), ())),
                            preferred_element_type=jnp.float32)
    h = jnp.maximum(h + b1_ref[...], 0.0)
    out = lax.dot_general(w2_ref[...], h, (((1,), (0,)), ((), ())),
                          preferred_element_type=jnp.float32)
    out_ref[...] = out + ub_ref[...] + ib_ref[...]


def _mlp(uzt, vzt, ub, ib, w1, b1, w2):
    grid = (BATCH // _BLK,)
    col_blk = lambda i: (0, i)
    w_blk = lambda i: (0, 0)
    return pl.pallas_call(
        _mlp_body,
        grid=grid,
        in_specs=[
            pl.BlockSpec((EMB_K, _BLK), col_blk),
            pl.BlockSpec((EMB_K, _BLK), col_blk),
            pl.BlockSpec((1, _BLK), col_blk),
            pl.BlockSpec((1, _BLK), col_blk),
            pl.BlockSpec((EMB_K, 2 * EMB_K), w_blk),
            pl.BlockSpec((EMB_K, 1), w_blk),
            pl.BlockSpec((1, EMB_K), w_blk),
        ],
        out_specs=pl.BlockSpec((1, _BLK), col_blk),
        out_shape=jax.ShapeDtypeStruct((1, BATCH), jnp.float32),
    )(uzt, vzt, ub, ib, w1, b1, w2)


def kernel(x, W, H, lin1_w, lin1_b, lin2_w, user_bias, item_bias):
    xt = x.T.reshape(-1)
    wtp = jnp.pad(W.T, ((0, 0), (0, _TW - W.shape[0])))
    htp = jnp.pad(H.T, ((0, 0), (0, _TW - H.shape[0])))
    ubf = user_bias.T.reshape(-1)
    ibf = item_bias.T.reshape(-1)
    uzt, vzt, ubg, ibg = _make_sc_gather()(xt, wtp, htp, ubf, ibf)
    out = _mlp(uzt, vzt, ubg.reshape(1, BATCH), ibg.reshape(1, BATCH),
               lin1_w, lin1_b.reshape(EMB_K, 1), lin2_w)
    return out.reshape(BATCH, 1)


# speedup vs baseline: 2.4427x; 1.0497x over previous
import functools

import jax
import jax.numpy as jnp
from jax import lax
from jax.experimental import pallas as pl
from jax.experimental.pallas import tpu as pltpu
from jax.experimental.pallas import tpu_sc as plsc

BATCH = 16384
EMB_K = 16

_NC, _NS = 2, 16
_BPT = BATCH // _NS
_L = 16
_TW = 100096
_ZCH = 2048


@functools.cache
def _make_sc_gather():
    mesh = plsc.VectorSubcoreMesh(core_axis_name="c", subcore_axis_name="s")

    @functools.partial(
        pl.kernel,
        mesh=mesh,
        compiler_params=pltpu.CompilerParams(use_tc_tiling_on_sc=False,
                                             needs_layout_passes=False),
        out_type=[
            jax.ShapeDtypeStruct((EMB_K, BATCH), jnp.float32),
            jax.ShapeDtypeStruct((EMB_K, BATCH), jnp.float32),
            jax.ShapeDtypeStruct((BATCH,), jnp.float32),
            jax.ShapeDtypeStruct((BATCH,), jnp.float32),
        ],
        scratch_types=[
            pltpu.VMEM((_TW,), jnp.float32),
            pltpu.VMEM((BATCH,), jnp.int32),
            pltpu.VMEM((2, _ZCH), jnp.float32),
            pltpu.VMEM((_BPT,), jnp.float32),
            pltpu.SemaphoreType.DMA,
            pltpu.SemaphoreType.DMA,
        ],
    )
    def gather_kernel(xt_hbm, tabs_hbm, ub_hbm, ib_hbm,
                      uzt_out, vzt_out, ubg_out, ibg_out,
                      tv, idx_v, zrow, br_v, sem, wsem):
        cid = lax.axis_index("c")
        sid = lax.axis_index("s")

        def side(trow, bias_hbm, xoff, zt_out, bg_out):
            stage_cp = pltpu.async_copy(tabs_hbm.at[trow], tv, sem)
            pltpu.async_copy(
                xt_hbm.at[pl.ds(xoff, BATCH)], idx_v, sem).wait()

            gb = sid * _BPT
            bias_cps = [
                pltpu.async_copy(
                    bias_hbm.at[idx_v.at[pl.ds(gb + i * 128, 128)]],
                    br_v.at[pl.ds(i * 128, 128)], sem)
                for i in range(_BPT // 128)
            ]
            stage_cp.wait()

            def chunk(j, c):
                b = j & 1
                pl.when(j >= 2)(lambda: pltpu.make_async_copy(
                    zrow.at[b], zt_out.at[sid, pl.ds(0, _ZCH)], wsem).wait())
                for g in range(_ZCH // _L):
                    s = pl.ds(g * _L, _L)
                    u = idx_v[pl.ds(j * _ZCH + g * _L, _L)]
                    zrow[b, s] = plsc.load_gather(tv, [u])
                pltpu.async_copy(
                    zrow.at[b], zt_out.at[sid, pl.ds(j * _ZCH, _ZCH)], wsem)
                return c

            lax.fori_loop(0, BATCH // _ZCH, chunk, 0)
            for _ in range(2):
                pltpu.make_async_copy(
                    zrow.at[0], zt_out.at[sid, pl.ds(0, _ZCH)], wsem).wait()

            for cp in bias_cps:
                cp.wait()
            pltpu.sync_copy(br_v, bg_out.at[pl.ds(gb, _BPT)])

        @pl.when(cid == 0)
        def _():
            side(sid, ub_hbm, 0, uzt_out, ubg_out)

        @pl.when(cid == 1)
        def _():
            side(sid + EMB_K, ib_hbm, BATCH, vzt_out, ibg_out)

    return gather_kernel


_BLK = 4096


def _mlp_body(uzt_ref, vzt_ref, ub_ref, ib_ref, w1_ref, b1_ref, w2_ref,
              out_ref):
    uzt = uzt_ref[...]
    vzt = vzt_ref[...]
    w1 = w1_ref[...]
    h = lax.dot_general(w1[:, :EMB_K], uzt, (((1,), (0,)), ((), ())),
                        preferred_element_type=jnp.float32)
    h = h + lax.dot_general(w1[:, EMB_K:], vzt, (((1,), (0,)), ((), ())),
                            preferred_element_type=jnp.float32)
    h = jnp.maximum(h + b1_ref[...], 0.0)
    out = lax.dot_general(w2_ref[...], h, (((1,), (0,)), ((), ())),
                          preferred_element_type=jnp.float32)
    out_ref[...] = out + ub_ref[...] + ib_ref[...]


def _mlp(uzt, vzt, ub, ib, w1, b1, w2):
    grid = (BATCH // _BLK,)
    col_blk = lambda i: (0, i)
    w_blk = lambda i: (0, 0)
    return pl.pallas_call(
        _mlp_body,
        grid=grid,
        in_specs=[
            pl.BlockSpec((EMB_K, _BLK), col_blk),
            pl.BlockSpec((EMB_K, _BLK), col_blk),
            pl.BlockSpec((1, _BLK), col_blk),
            pl.BlockSpec((1, _BLK), col_blk),
            pl.BlockSpec((EMB_K, 2 * EMB_K), w_blk),
            pl.BlockSpec((EMB_K, 1), w_blk),
            pl.BlockSpec((1, EMB_K), w_blk),
        ],
        out_specs=pl.BlockSpec((1, _BLK), col_blk),
        out_shape=jax.ShapeDtypeStruct((1, BATCH), jnp.float32),
    )(uzt, vzt, ub, ib, w1, b1, w2)


def kernel(x, W, H, lin1_w, lin1_b, lin2_w, user_bias, item_bias):
    xt = x.T.reshape(-1)
    tabs = jnp.pad(jnp.concatenate([W.T, H.T], axis=0),
                   ((0, 0), (0, _TW - W.shape[0])))
    ubf = user_bias.T.reshape(-1)
    ibf = item_bias.T.reshape(-1)
    uzt, vzt, ubg, ibg = _make_sc_gather()(xt, tabs, ubf, ibf)
    out = _mlp(uzt, vzt, ubg.reshape(1, BATCH), ibg.reshape(1, BATCH),
               lin1_w, lin1_b.reshape(EMB_K, 1), lin2_w)
    return out.reshape(BATCH, 1)
